# baseline (speedup 1.0000x reference)
import functools

import jax
import jax.numpy as jnp
from jax import lax
from jax.experimental import pallas as pl
from jax.experimental.pallas import tpu as pltpu
from jax.experimental.pallas import tpu_sc as plsc

N = 10000
E = 320000
D = 128

NC = 2
NS = 16
NW = NC * NS
NPT = 313
NPAD = NW * NPT
CHUNK = 4000
NCHUNK = E // CHUNK
G = 128
MCAP = 4096
NEG = float("-inf")
EPS = 1e-5


def _segmax_body(y_hbm, src_hbm, dst_hbm, out_hbm,
                 schunk, dchunk, msrc, mdst, rows, acc, sem):
    wid = lax.axis_index("s") * NC + lax.axis_index("c")
    lo = wid * NPT

    def init_acc(i, _):
        acc[pl.ds(i * 16, 16)] = jnp.full((16,), NEG, jnp.float32)
        return 0
    lax.fori_loop(0, (NPT + 1) * D // 16, init_acc, 0)

    def init_msrc(i, _):
        msrc[pl.ds(i * 16, 16)] = jnp.zeros((16,), jnp.int32)
        return 0
    lax.fori_loop(0, MCAP // 16, init_msrc, 0)

    def chunk_body(c, _):
        with jax.named_scope("edma"):
            pltpu.sync_copy(src_hbm.at[pl.ds(c * CHUNK, CHUNK)], schunk)
            pltpu.sync_copy(dst_hbm.at[pl.ds(c * CHUNK, CHUNK)], dchunk)

        def filt(i, cnt):
            dv = dchunk[pl.ds(i * 16, 16)]
            sv = schunk[pl.ds(i * 16, 16)]
            dl = dv - lo
            m = (dl >= 0) & (dl < NPT)
            pos = cnt + plsc.cumsum(m.astype(jnp.int32)) - 1
            plsc.store_scatter(msrc, [pos], sv, mask=m)
            plsc.store_scatter(mdst, [pos], dl, mask=m)
            return cnt + jnp.max(plsc.all_reduce_population_count(m))
        with jax.named_scope("filt"):
            cnt = lax.fori_loop(0, CHUNK // 16, filt, jnp.int32(0))

        mdst[pl.ds(cnt, 16)] = jnp.full((16,), NPT, jnp.int32)

        ng = (cnt + (G - 1)) // G

        def gbody(g, _):
            with jax.named_scope("gath"):
                pltpu.async_copy(y_hbm.at[msrc.at[pl.ds(g * G, G)]],
                                 rows.at[pl.ds(0, G)], sem).wait()
            ec = jnp.minimum(G, cnt - g * G)

            def kbody(k, _):
                dvec = mdst[pl.ds(g * G + k * 16, 16)]
                for l in range(16):
                    base = dvec[l] * D
                    el = k * 16 + l
                    for j in range(D // 16):
                        a = acc[pl.ds(base + j * 16, 16)]
                        r = rows[el, pl.ds(j * 16, 16)]
                        acc[pl.ds(base + j * 16, 16)] = jnp.maximum(a, r)
                return 0
            with jax.named_scope("upd"):
                lax.fori_loop(0, (ec + 15) // 16, kbody, 0)
            return 0
        lax.fori_loop(0, ng, gbody, 0)
        return 0
    lax.fori_loop(0, NCHUNK, chunk_body, 0)

    pltpu.sync_copy(acc.at[pl.ds(0, NPT * D)],
                    out_hbm.at[pl.ds(lo * D, NPT * D)])


@functools.lru_cache(maxsize=1)
def _make_sc_segmax():
    return functools.partial(
        pl.kernel,
        compiler_params=pltpu.CompilerParams(needs_layout_passes=False),
        mesh=plsc.VectorSubcoreMesh(core_axis_name="c", subcore_axis_name="s"),
        out_type=jax.ShapeDtypeStruct((NPAD * D,), jnp.float32),
        scratch_types=[
            pltpu.VMEM((CHUNK,), jnp.int32),
            pltpu.VMEM((CHUNK,), jnp.int32),
            pltpu.VMEM((MCAP,), jnp.int32),
            pltpu.VMEM((MCAP,), jnp.int32),
            pltpu.VMEM((G + 16, D), jnp.float32),
            pltpu.VMEM(((NPT + 1) * D,), jnp.float32),
            pltpu.SemaphoreType.DMA,
        ],
    )(_segmax_body)


def _sc_segmax(y, src, dst):
    return _make_sc_segmax()(y, src, dst)


def _bn(h, g, be):
    mu = jnp.mean(h, axis=0, keepdims=True)
    var = jnp.mean((h - mu) ** 2, axis=0, keepdims=True)
    return g * (h - mu) / jnp.sqrt(var + EPS) + be


def _tc_pre_body(x_ref, W1_ref, b1_ref, Wl_ref, bl_ref, gl_ref, bel_ref,
                 y1_ref, skip_ref):
    x = x_ref[...]
    xw = jnp.dot(x, Wl_ref[...], preferred_element_type=jnp.float32) \
        + bl_ref[...]
    skip_ref[...] = _bn(xw, gl_ref[...], bel_ref[...])
    y1_ref[...] = jnp.dot(x, W1_ref[...],
                          preferred_element_type=jnp.float32) + b1_ref[...]


def _tc_mid_body(agg_ref, g1_ref, be1_ref, W2_ref, b2_ref, y2_ref):
    h = agg_ref[...]
    h = jnp.where(h == NEG, 0.0, h)
    h = jnp.maximum(_bn(h, g1_ref[...], be1_ref[...]), 0.0)
    y2_ref[...] = jnp.dot(h, W2_ref[...],
                          preferred_element_type=jnp.float32) + b2_ref[...]


def _tc_post_body(agg_ref, skip_ref, g2_ref, be2_ref, out_ref):
    h = agg_ref[...]
    h = jnp.where(h == NEG, 0.0, h)
    h = _bn(h, g2_ref[...], be2_ref[...])
    out_ref[...] = jnp.maximum(h + skip_ref[...], 0.0)


_tc_pre = pl.pallas_call(
    _tc_pre_body,
    out_shape=[jax.ShapeDtypeStruct((N, D), jnp.float32),
               jax.ShapeDtypeStruct((N, D), jnp.float32)],
)

_tc_mid = pl.pallas_call(
    _tc_mid_body,
    out_shape=jax.ShapeDtypeStruct((N, D), jnp.float32),
)

_tc_post = pl.pallas_call(
    _tc_post_body,
    out_shape=jax.ShapeDtypeStruct((N, D), jnp.float32),
)


def kernel(x, edge_index, W1, b1, W2, b2, Wl, bl, g1, be1, g2, be2, gl, bel):
    src = edge_index[0]
    dst = edge_index[1]
    r = lambda v: v.reshape(1, D)
    y1, skip = _tc_pre(x, W1, r(b1), Wl, r(bl), r(gl), r(bel))
    agg1 = _sc_segmax(y1, src, dst).reshape(NPAD, D)[:N]
    y2 = _tc_mid(agg1, r(g1), r(be1), W2, r(b2))
    agg2 = _sc_segmax(y2, src, dst).reshape(NPAD, D)[:N]
    return _tc_post(agg2, skip, r(g2), r(be2))


# speedup vs baseline: 5.7569x; 5.7569x over previous
import functools

import jax
import jax.numpy as jnp
from jax import lax
from jax.experimental import pallas as pl
from jax.experimental.pallas import tpu as pltpu
from jax.experimental.pallas import tpu_sc as plsc

N = 10000
E = 320000
D = 128

NC = 2
NS = 16
NW = NC * NS
NPT = 313
NPAD = NW * NPT
CHUNK = 2000
NCHUNK = E // CHUNK
CSTRIDE = NCHUNK // NW
G = 128
QD = 4
MCAP = 4096
NEG = float("-inf")
EPS = 1e-5


def _segmax_body(y_hbm, src_hbm, dst_hbm, out_hbm,
                 schunk0, schunk1, dchunk0, dchunk1, msrc, mdst,
                 rows0, rows1, rows2, rows3, acc,
                 semc, semg0, semg1, semg2, semg3):
    schunk = (schunk0, schunk1)
    dchunk = (dchunk0, dchunk1)
    rows = (rows0, rows1, rows2, rows3)
    semg = (semg0, semg1, semg2, semg3)

    cid = lax.axis_index("c")
    sid = lax.axis_index("s")
    wid = sid * NC + cid
    lo = wid * NPT
    cbase = wid * CSTRIDE

    def init_acc(i, _):
        acc[pl.ds(i * 16, 16)] = jnp.full((16,), NEG, jnp.float32)
        return 0
    lax.fori_loop(0, (NPT + 1) * D // 16, init_acc, 0)

    def init_msrc(i, _):
        msrc[pl.ds(i * 16, 16)] = jnp.zeros((16,), jnp.int32)
        return 0
    lax.fori_loop(0, MCAP // 16, init_msrc, 0)

    def fire_chunk(step, b):
        c = lax.rem(cbase + step, NCHUNK)
        pltpu.async_copy(src_hbm.at[pl.ds(c * CHUNK, CHUNK)],
                         schunk[b], semc)
        pltpu.async_copy(dst_hbm.at[pl.ds(c * CHUNK, CHUNK)],
                         dchunk[b], semc)

    def wait_chunk(b):
        pltpu.make_async_copy(src_hbm.at[pl.ds(0, CHUNK)],
                              schunk[b], semc).wait()
        pltpu.make_async_copy(dst_hbm.at[pl.ds(0, CHUNK)],
                              dchunk[b], semc).wait()

    def fire_gather(bq):
        off = lax.rem(bq * G, MCAP)
        idx = msrc.at[pl.ds(off, G)]
        for b in range(QD):
            @pl.when(lax.rem(bq, QD) == b)
            def _():
                pltpu.async_copy(y_hbm.at[idx], rows[b], semg[b])

    def wait_update(bq, cnt):
        ec = jnp.minimum(G, cnt - bq * G)
        moff = lax.rem(bq * G, MCAP)
        for b in range(QD):
            @pl.when(lax.rem(bq, QD) == b)
            def _():
                idx = msrc.at[pl.ds(moff, G)]
                rb = rows[b]
                pltpu.make_async_copy(y_hbm.at[idx], rb, semg[b]).wait()

                def kbody(k, _):
                    dvec = mdst[pl.ds(moff + k * 16, 16)]
                    for l in range(16):
                        base = dvec[l] * D
                        el = jnp.minimum(k * 16 + l, G - 1)
                        for j in range(D // 16):
                            a = acc[pl.ds(base + j * 16, 16)]
                            r = rb[el, pl.ds(j * 16, 16)]
                            acc[pl.ds(base + j * 16, 16)] = jnp.maximum(a, r)
                    return 0
                lax.fori_loop(0, (ec + 15) // 16, kbody, 0)

    fire_chunk(0, 0)

    def chunk_pair(cp, st):
        carry = st
        for b in range(2):
            cnt_in, fired_in = carry
            step = cp * 2 + b
            wait_chunk(b)

            @pl.when(step + 1 < NCHUNK)
            def _():
                fire_chunk(step + 1, 1 - b)

            sb = schunk[b]
            db = dchunk[b]

            def filt(i, cnt):
                dv = db[pl.ds(i * 16, 16)]
                sv = sb[pl.ds(i * 16, 16)]
                dl = dv - lo
                m = (dl >= 0) & (dl < NPT)
                pos = cnt + plsc.cumsum(m.astype(jnp.int32)) - 1
                posm = lax.rem(pos, MCAP)
                plsc.store_scatter(msrc, [posm], sv, mask=m)
                plsc.store_scatter(mdst, [posm], dl, mask=m)
                return pos[15] + 1
            cnt = lax.fori_loop(0, CHUNK // 16, filt, cnt_in)

            avail = cnt // G

            def bloop(bq, _):
                @pl.when(bq >= QD)
                def _():
                    wait_update(bq - QD, cnt)
                fire_gather(bq)
                return 0
            lax.fori_loop(fired_in, avail, bloop, 0)
            carry = (cnt, avail)
        return carry

    cnt, fired = lax.fori_loop(0, NCHUNK // 2, chunk_pair,
                               (jnp.int32(0), jnp.int32(0)))

    mdst[pl.ds(lax.rem(cnt, MCAP), 16)] = jnp.full((16,), NPT, jnp.int32)
    total_b = (cnt + (G - 1)) // G
    have_partial = total_b > fired

    @pl.when(have_partial & (fired >= QD))
    def _():
        wait_update(fired - QD, cnt)

    @pl.when(have_partial)
    def _():
        fire_gather(fired)

    u0 = jnp.maximum(fired - QD, 0)
    u0 = jnp.where(have_partial & (fired >= QD), u0 + 1, u0)

    def drain(bq, _):
        wait_update(bq, cnt)
        return 0
    lax.fori_loop(u0, total_b, drain, 0)

    pltpu.sync_copy(acc.at[pl.ds(0, NPT * D)],
                    out_hbm.at[pl.ds(lo * D, NPT * D)])


@functools.lru_cache(maxsize=1)
def _make_sc_segmax():
    return functools.partial(
        pl.kernel,
        compiler_params=pltpu.CompilerParams(needs_layout_passes=False),
        mesh=plsc.VectorSubcoreMesh(core_axis_name="c", subcore_axis_name="s"),
        out_type=jax.ShapeDtypeStruct((NPAD * D,), jnp.float32),
        scratch_types=[
            pltpu.VMEM((CHUNK,), jnp.int32),
            pltpu.VMEM((CHUNK,), jnp.int32),
            pltpu.VMEM((CHUNK,), jnp.int32),
            pltpu.VMEM((CHUNK,), jnp.int32),
            pltpu.VMEM((MCAP,), jnp.int32),
            pltpu.VMEM((MCAP,), jnp.int32),
            pltpu.VMEM((G, D), jnp.float32),
            pltpu.VMEM((G, D), jnp.float32),
            pltpu.VMEM((G, D), jnp.float32),
            pltpu.VMEM((G, D), jnp.float32),
            pltpu.VMEM(((NPT + 1) * D,), jnp.float32),
            pltpu.SemaphoreType.DMA,
            pltpu.SemaphoreType.DMA,
            pltpu.SemaphoreType.DMA,
            pltpu.SemaphoreType.DMA,
            pltpu.SemaphoreType.DMA,
        ],
    )(_segmax_body)


def _sc_segmax(y, src, dst):
    return _make_sc_segmax()(y, src, dst)


def _bn(h, g, be):
    mu = jnp.mean(h, axis=0, keepdims=True)
    var = jnp.mean((h - mu) ** 2, axis=0, keepdims=True)
    return g * (h - mu) / jnp.sqrt(var + EPS) + be


def _tc_pre_body(x_ref, W1_ref, b1_ref, Wl_ref, bl_ref, gl_ref, bel_ref,
                 y1_ref, skip_ref):
    x = x_ref[...]
    xw = jnp.dot(x, Wl_ref[...], preferred_element_type=jnp.float32) \
        + bl_ref[...]
    skip_ref[...] = _bn(xw, gl_ref[...], bel_ref[...])
    y1_ref[...] = jnp.dot(x, W1_ref[...],
                          preferred_element_type=jnp.float32) + b1_ref[...]


def _tc_mid_body(agg_ref, g1_ref, be1_ref, W2_ref, b2_ref, y2_ref):
    h = agg_ref[...]
    h = jnp.where(h == NEG, 0.0, h)
    h = jnp.maximum(_bn(h, g1_ref[...], be1_ref[...]), 0.0)
    y2_ref[...] = jnp.dot(h, W2_ref[...],
                          preferred_element_type=jnp.float32) + b2_ref[...]


def _tc_post_body(agg_ref, skip_ref, g2_ref, be2_ref, out_ref):
    h = agg_ref[...]
    h = jnp.where(h == NEG, 0.0, h)
    h = _bn(h, g2_ref[...], be2_ref[...])
    out_ref[...] = jnp.maximum(h + skip_ref[...], 0.0)


_tc_pre = pl.pallas_call(
    _tc_pre_body,
    out_shape=[jax.ShapeDtypeStruct((N, D), jnp.float32),
               jax.ShapeDtypeStruct((N, D), jnp.float32)],
)

_tc_mid = pl.pallas_call(
    _tc_mid_body,
    out_shape=jax.ShapeDtypeStruct((N, D), jnp.float32),
)

_tc_post = pl.pallas_call(
    _tc_post_body,
    out_shape=jax.ShapeDtypeStruct((N, D), jnp.float32),
)


def kernel(x, edge_index, W1, b1, W2, b2, Wl, bl, g1, be1, g2, be2, gl, bel):
    src = edge_index[0]
    dst = edge_index[1]
    r = lambda v: v.reshape(1, D)
    y1, skip = _tc_pre(x, W1, r(b1), Wl, r(bl), r(gl), r(bel))
    agg1 = _sc_segmax(y1, src, dst).reshape(NPAD, D)[:N]
    y2 = _tc_mid(agg1, r(g1), r(be1), W2, r(b2))
    agg2 = _sc_segmax(y2, src, dst).reshape(NPAD, D)[:N]
    return _tc_post(agg2, skip, r(g2), r(be2))


# speedup vs baseline: 6.3108x; 1.0962x over previous
import functools

import jax
import jax.numpy as jnp
from jax import lax
from jax.experimental import pallas as pl
from jax.experimental.pallas import tpu as pltpu
from jax.experimental.pallas import tpu_sc as plsc

N = 10000
E = 320000
D = 128

NC = 2
NS = 16
NW = NC * NS
NPT = 313
NPAD = NW * NPT
CHUNK = 2000
NCHUNK = E // CHUNK
CSTRIDE = NCHUNK // NW
G = 128
QD = 4
MCAP = 4096
NEG = float("-inf")
EPS = 1e-5


def _segmax_body(y_hbm, src_hbm, dst_hbm, out_hbm,
                 schunk0, schunk1, dchunk0, dchunk1, msrc, mdst,
                 rows0, rows1, rows2, rows3, acc,
                 semc, semg0, semg1, semg2, semg3):
    schunk = (schunk0, schunk1)
    dchunk = (dchunk0, dchunk1)
    rows = (rows0, rows1, rows2, rows3)
    semg = (semg0, semg1, semg2, semg3)

    cid = lax.axis_index("c")
    sid = lax.axis_index("s")
    wid = sid * NC + cid
    lo = wid * NPT
    cbase = wid * CSTRIDE

    def init_acc(i, _):
        acc[pl.ds(i * 16, 16)] = jnp.full((16,), NEG, jnp.float32)
        return 0
    lax.fori_loop(0, (NPT + 1) * D // 16, init_acc, 0)

    def init_msrc(i, _):
        msrc[pl.ds(i * 16, 16)] = jnp.zeros((16,), jnp.int32)
        return 0
    lax.fori_loop(0, MCAP // 16, init_msrc, 0)

    def fire_chunk(step, b):
        c = lax.rem(cbase + step, NCHUNK)
        pltpu.async_copy(src_hbm.at[pl.ds(c * CHUNK, CHUNK)],
                         schunk[b], semc)
        pltpu.async_copy(dst_hbm.at[pl.ds(c * CHUNK, CHUNK)],
                         dchunk[b], semc)

    def wait_chunk(b):
        pltpu.make_async_copy(src_hbm.at[pl.ds(0, CHUNK)],
                              schunk[b], semc).wait()
        pltpu.make_async_copy(dst_hbm.at[pl.ds(0, CHUNK)],
                              dchunk[b], semc).wait()

    def fire_gather(bq):
        off = pl.multiple_of(jnp.bitwise_and(bq * G, MCAP - 1), G)
        idx = msrc.at[pl.ds(off, G)]
        for b in range(QD):
            @pl.when(jnp.bitwise_and(bq, QD - 1) == b)
            def _():
                pltpu.async_copy(y_hbm.at[idx], rows[b], semg[b])

    def wait_update(bq, cnt):
        ec = jnp.minimum(G, cnt - bq * G)
        moff = pl.multiple_of(jnp.bitwise_and(bq * G, MCAP - 1), G)
        for b in range(QD):
            @pl.when(jnp.bitwise_and(bq, QD - 1) == b)
            def _():
                idx = msrc.at[pl.ds(moff, G)]
                rb = rows[b]
                pltpu.make_async_copy(y_hbm.at[idx], rb, semg[b]).wait()

                def kbody(k, _):
                    dvec = mdst[pl.ds(moff + k * 16, 16)]
                    for l in range(16):
                        base = dvec[l] * D
                        el = jnp.minimum(k * 16 + l, G - 1)
                        for j in range(D // 16):
                            a = acc[pl.ds(base + j * 16, 16)]
                            r = rb[el, pl.ds(j * 16, 16)]
                            acc[pl.ds(base + j * 16, 16)] = jnp.maximum(a, r)
                    return 0
                lax.fori_loop(0, (ec + 15) // 16, kbody, 0)

    fire_chunk(0, 0)

    def chunk_pair(cp, st):
        carry = st
        for b in range(2):
            cnt_in, fired_in = carry
            step = cp * 2 + b
            wait_chunk(b)

            @pl.when(step + 1 < NCHUNK)
            def _():
                fire_chunk(step + 1, 1 - b)

            sb = schunk[b]
            db = dchunk[b]

            def filt(i, cnt):
                dv0 = db[pl.ds(i * 32, 16)]
                sv0 = sb[pl.ds(i * 32, 16)]
                dv1 = db[pl.ds(i * 32 + 16, 16)]
                sv1 = sb[pl.ds(i * 32 + 16, 16)]
                dl0 = dv0 - lo
                dl1 = dv1 - lo
                m0 = (dl0 >= 0) & (dl0 < NPT)
                m1 = (dl1 >= 0) & (dl1 < NPT)
                cs0 = plsc.cumsum(m0.astype(jnp.int32))
                cs1 = plsc.cumsum(m1.astype(jnp.int32))
                cnt1 = cnt + cs0[15]
                pos0 = jnp.bitwise_and(cnt + cs0 - 1, MCAP - 1)
                pos1 = jnp.bitwise_and(cnt1 + cs1 - 1, MCAP - 1)
                plsc.store_scatter(msrc, [pos0], sv0, mask=m0)
                plsc.store_scatter(mdst, [pos0], dl0, mask=m0)
                plsc.store_scatter(msrc, [pos1], sv1, mask=m1)
                plsc.store_scatter(mdst, [pos1], dl1, mask=m1)
                return cnt1 + cs1[15]
            cnt = lax.fori_loop(0, CHUNK // 32, filt, cnt_in)

            avail = cnt // G

            def bloop(bq, _):
                @pl.when(bq >= QD)
                def _():
                    wait_update(bq - QD, cnt)
                fire_gather(bq)
                return 0
            lax.fori_loop(fired_in, avail, bloop, 0)
            carry = (cnt, avail)
        return carry

    cnt, fired = lax.fori_loop(0, NCHUNK // 2, chunk_pair,
                               (jnp.int32(0), jnp.int32(0)))

    mdst[pl.ds(jnp.bitwise_and(cnt, MCAP - 1), 16)] = jnp.full((16,), NPT,
                                                           jnp.int32)
    total_b = (cnt + (G - 1)) // G
    have_partial = total_b > fired

    @pl.when(have_partial & (fired >= QD))
    def _():
        wait_update(fired - QD, cnt)

    @pl.when(have_partial)
    def _():
        fire_gather(fired)

    u0 = jnp.maximum(fired - QD, 0)
    u0 = jnp.where(have_partial & (fired >= QD), u0 + 1, u0)

    def drain(bq, _):
        wait_update(bq, cnt)
        return 0
    lax.fori_loop(u0, total_b, drain, 0)

    pltpu.sync_copy(acc.at[pl.ds(0, NPT * D)],
                    out_hbm.at[pl.ds(lo * D, NPT * D)])


@functools.lru_cache(maxsize=1)
def _make_sc_segmax():
    return functools.partial(
        pl.kernel,
        compiler_params=pltpu.CompilerParams(needs_layout_passes=False),
        mesh=plsc.VectorSubcoreMesh(core_axis_name="c", subcore_axis_name="s"),
        out_type=jax.ShapeDtypeStruct((NPAD * D,), jnp.float32),
        scratch_types=[
            pltpu.VMEM((CHUNK,), jnp.int32),
            pltpu.VMEM((CHUNK,), jnp.int32),
            pltpu.VMEM((CHUNK,), jnp.int32),
            pltpu.VMEM((CHUNK,), jnp.int32),
            pltpu.VMEM((MCAP,), jnp.int32),
            pltpu.VMEM((MCAP,), jnp.int32),
            pltpu.VMEM((G, D), jnp.float32),
            pltpu.VMEM((G, D), jnp.float32),
            pltpu.VMEM((G, D), jnp.float32),
            pltpu.VMEM((G, D), jnp.float32),
            pltpu.VMEM(((NPT + 1) * D,), jnp.float32),
            pltpu.SemaphoreType.DMA,
            pltpu.SemaphoreType.DMA,
            pltpu.SemaphoreType.DMA,
            pltpu.SemaphoreType.DMA,
            pltpu.SemaphoreType.DMA,
        ],
    )(_segmax_body)


def _sc_segmax(y, src, dst):
    return _make_sc_segmax()(y, src, dst)


def _bn(h, g, be):
    mu = jnp.mean(h, axis=0, keepdims=True)
    var = jnp.mean((h - mu) ** 2, axis=0, keepdims=True)
    return g * (h - mu) / jnp.sqrt(var + EPS) + be


def _tc_pre_body(x_ref, W1_ref, b1_ref, Wl_ref, bl_ref, gl_ref, bel_ref,
                 y1_ref, skip_ref):
    x = x_ref[...]
    xw = jnp.dot(x, Wl_ref[...], preferred_element_type=jnp.float32) \
        + bl_ref[...]
    skip_ref[...] = _bn(xw, gl_ref[...], bel_ref[...])
    y1_ref[...] = jnp.dot(x, W1_ref[...],
                          preferred_element_type=jnp.float32) + b1_ref[...]


def _tc_mid_body(agg_ref, g1_ref, be1_ref, W2_ref, b2_ref, y2_ref):
    h = agg_ref[...]
    h = jnp.where(h == NEG, 0.0, h)
    h = jnp.maximum(_bn(h, g1_ref[...], be1_ref[...]), 0.0)
    y2_ref[...] = jnp.dot(h, W2_ref[...],
                          preferred_element_type=jnp.float32) + b2_ref[...]


def _tc_post_body(agg_ref, skip_ref, g2_ref, be2_ref, out_ref):
    h = agg_ref[...]
    h = jnp.where(h == NEG, 0.0, h)
    h = _bn(h, g2_ref[...], be2_ref[...])
    out_ref[...] = jnp.maximum(h + skip_ref[...], 0.0)


_tc_pre = pl.pallas_call(
    _tc_pre_body,
    out_shape=[jax.ShapeDtypeStruct((N, D), jnp.float32),
               jax.ShapeDtypeStruct((N, D), jnp.float32)],
)

_tc_mid = pl.pallas_call(
    _tc_mid_body,
    out_shape=jax.ShapeDtypeStruct((N, D), jnp.float32),
)

_tc_post = pl.pallas_call(
    _tc_post_body,
    out_shape=jax.ShapeDtypeStruct((N, D), jnp.float32),
)


def kernel(x, edge_index, W1, b1, W2, b2, Wl, bl, g1, be1, g2, be2, gl, bel):
    src = edge_index[0]
    dst = edge_index[1]
    r = lambda v: v.reshape(1, D)
    y1, skip = _tc_pre(x, W1, r(b1), Wl, r(bl), r(gl), r(bel))
    agg1 = _sc_segmax(y1, src, dst).reshape(NPAD, D)[:N]
    y2 = _tc_mid(agg1, r(g1), r(be1), W2, r(b2))
    agg2 = _sc_segmax(y2, src, dst).reshape(NPAD, D)[:N]
    return _tc_post(agg2, skip, r(g2), r(be2))
